# baseline (speedup 1.0000x reference)
import functools

import jax
import jax.numpy as jnp
from jax import lax
from jax.experimental import pallas as pl
from jax.experimental.pallas import tpu as pltpu
from jax.experimental.pallas import tpu_sc as plsc

VOCAB = 1000000
EMBED = 32
HIDDEN = 64
OUT_VOCAB = 1000
BATCH = 4096
HIST = 200

PACK_S = 1 << 18
PACK_SHIFT = 18
PACK_MASK = PACK_S - 1
VIEW_ROWS = 4 * PACK_S

NC = 2
NS = 16
NW = NC * NS
B_PER_W = BATCH // NW
CB = 4
NCHUNK = B_PER_W // CB
CHUNK_IDX = CB * HIST
IDX_SPLITS = ((0, 128), (128, 72))


def _pool_body(texts_hbm, table_hbm, out_hbm,
               idx0, idx1, rows0, rows1, acc_v, sem0, sem1):
    wid = lax.axis_index("s") * NC + lax.axis_index("c")
    inv = jnp.float32(1.0 / HIST)
    idx_bufs = (idx0, idx1)
    rows_bufs = (rows0, rows1)
    sems = (sem0, sem1)

    def fire(g, buf):
        idx_v, rows_v, sem = idx_bufs[buf], rows_bufs[buf], sems[buf]
        cid = wid * NCHUNK + g
        pltpu.sync_copy(texts_hbm.at[pl.ds(cid * CHUNK_IDX, CHUNK_IDX)], idx_v)
        for t in range(CHUNK_IDX // 16):
            v = idx_v[pl.ds(16 * t, 16)]
            idx_v[pl.ds(16 * t, 16)] = ((v & PACK_MASK) << 2) | (v >> PACK_SHIFT)
        for c in range(CB):
            for off, ln in IDX_SPLITS:
                pltpu.async_copy(
                    table_hbm.at[idx_v.at[pl.ds(c * HIST + off, ln)]],
                    rows_v.at[pl.ds(c * HIST + off, ln)],
                    sem,
                )

    def drain(buf):
        rows_v, sem = rows_bufs[buf], sems[buf]
        for c in range(CB):
            for off, ln in IDX_SPLITS:
                pltpu.make_async_copy(
                    table_hbm.at[idx_bufs[buf].at[pl.ds(c * HIST + off, ln)]],
                    rows_v.at[pl.ds(c * HIST + off, ln)],
                    sem,
                ).wait()

    def compute(g, buf):
        rows_v = rows_bufs[buf]
        cid = wid * NCHUNK + g
        for c in range(CB):
            def sum_body(i, carry):
                ae, ao = carry
                r = c * HIST + i * 4
                for u in range(4):
                    we, wo = plsc.unpack(
                        rows_v[r + u, 0:EMBED],
                        format=plsc.PackFormat.INTERLEAVED,
                    )
                    ae = ae + we
                    ao = ao + wo
                return ae, ao

            ae, ao = lax.fori_loop(
                0, HIST // 4, sum_body,
                (jnp.zeros((16,), jnp.float32), jnp.zeros((16,), jnp.float32)),
            )
            acc_v[c, 0:16] = ae * inv
            acc_v[c, 16:32] = ao * inv
        pltpu.sync_copy(acc_v, out_hbm.at[pl.ds(cid * CB, CB)])

    fire(0, 0)

    def pair_body(g2, _):
        g = g2 * 2

        @pl.when(g + 1 < NCHUNK)
        def _():
            fire(g + 1, 1)

        drain(0)
        compute(g, 0)

        @pl.when(g + 2 < NCHUNK)
        def _():
            fire(g + 2, 0)

        drain(1)
        compute(g + 1, 1)
        return ()

    lax.fori_loop(0, NCHUNK // 2, pair_body, ())


def _sc_pool(texts_flat, table_view):
    mesh = plsc.VectorSubcoreMesh(core_axis_name="c", subcore_axis_name="s")
    f = pl.kernel(
        _pool_body,
        mesh=mesh,
        compiler_params=pltpu.CompilerParams(
            use_tc_tiling_on_sc=False, needs_layout_passes=False),
        out_type=jax.ShapeDtypeStruct((BATCH, EMBED), jnp.float32),
        scratch_types=[
            pltpu.VMEM((CHUNK_IDX,), jnp.int32),
            pltpu.VMEM((CHUNK_IDX,), jnp.int32),
            pltpu.VMEM((CHUNK_IDX, EMBED), jnp.bfloat16),
            pltpu.VMEM((CHUNK_IDX, EMBED), jnp.bfloat16),
            pltpu.VMEM((CB, EMBED), jnp.float32),
            pltpu.SemaphoreType.DMA,
            pltpu.SemaphoreType.DMA,
        ],
    )
    return f(texts_flat, table_view)


RL_VB = 2048
RL_QBLK = PACK_S // RL_VB
RL_INBLKS = (VOCAB + RL_VB - 1) // RL_VB


def _relayout_body(s0, s1, s2, s3, out_ref):
    x4 = jnp.concatenate([s0[...], s1[...], s2[...], s3[...]], axis=0)
    out_ref[...] = jnp.transpose(x4, (1, 0)).astype(jnp.bfloat16)


def _quarter_spec(a):
    def imap(i):
        return (0, jnp.minimum(a * RL_QBLK + i, RL_INBLKS - 1))
    return pl.BlockSpec((EMBED, RL_VB), imap)


def _tc_relayout(table):
    packed = pl.pallas_call(
        _relayout_body,
        grid=(RL_QBLK,),
        in_specs=[_quarter_spec(a) for a in range(4)],
        out_specs=pl.BlockSpec((RL_VB, 4 * EMBED), lambda i: (i, 0)),
        out_shape=jax.ShapeDtypeStruct((PACK_S, 4 * EMBED), jnp.bfloat16),
    )(table.T, table.T, table.T, table.T)
    return packed.reshape(VIEW_ROWS, EMBED)


def _mlp_body(x_ref, w1_ref, b1_ref, w2_ref, b2_ref, o_ref):
    h = jnp.dot(x_ref[...], w1_ref[...], preferred_element_type=jnp.float32)
    h = jnp.maximum(h + b1_ref[...], 0.0)
    o = jnp.dot(h, w2_ref[...], preferred_element_type=jnp.float32)
    o_ref[...] = o + b2_ref[...]


def _tc_mlp(pooled, W1, b1, W2, b2):
    BM = 512
    grid = (BATCH // BM,)
    return pl.pallas_call(
        _mlp_body,
        grid=grid,
        in_specs=[
            pl.BlockSpec((BM, EMBED), lambda i: (i, 0)),
            pl.BlockSpec((EMBED, HIDDEN), lambda i: (0, 0)),
            pl.BlockSpec((1, HIDDEN), lambda i: (0, 0)),
            pl.BlockSpec((HIDDEN, OUT_VOCAB), lambda i: (0, 0)),
            pl.BlockSpec((1, OUT_VOCAB), lambda i: (0, 0)),
        ],
        out_specs=pl.BlockSpec((BM, OUT_VOCAB), lambda i: (i, 0)),
        out_shape=jax.ShapeDtypeStruct((BATCH, OUT_VOCAB), jnp.float32),
    )(pooled, W1, b1.reshape(1, HIDDEN), W2, b2.reshape(1, OUT_VOCAB))


_W1_PERM = tuple(range(0, EMBED, 2)) + tuple(range(1, EMBED, 2))


def kernel(texts, table, W1, b1, W2, b2):
    table_view = _tc_relayout(table)
    texts_flat = texts.astype(jnp.int32).reshape(BATCH * HIST)
    pooled = _sc_pool(texts_flat, table_view)
    w1p = W1[jnp.array(_W1_PERM), :]
    return _tc_mlp(pooled, w1p, b1, W2, b2)


# speedup vs baseline: 1.5177x; 1.5177x over previous
import functools

import jax
import jax.numpy as jnp
from jax import lax
from jax.experimental import pallas as pl
from jax.experimental.pallas import tpu as pltpu
from jax.experimental.pallas import tpu_sc as plsc

VOCAB = 1000000
EMBED = 32
HIDDEN = 64
OUT_VOCAB = 1000
BATCH = 4096
HIST = 200

PACK_S = 1 << 17
PACK_SHIFT = 17
PACK_MASK = PACK_S - 1
PACK_SLOTS = 8
VIEW_ROWS = PACK_SLOTS * PACK_S
ROW_I32 = EMBED // 2

NC = 2
NS = 16
NW = NC * NS
B_PER_W = BATCH // NW
CB = 4
NCHUNK = B_PER_W // CB
CHUNK_IDX = CB * HIST
IDX_SPLITS = ((0, 128), (128, 72))


def _pool_body(texts_hbm, table_hbm, out_hbm,
               idx0, idx1, rows0, rows1, acc_v, sem0, sem1):
    wid = lax.axis_index("s") * NC + lax.axis_index("c")
    inv = jnp.float32(1.0 / HIST)
    idx_bufs = (idx0, idx1)
    rows_bufs = (rows0, rows1)
    sems = (sem0, sem1)

    def fire(g, buf):
        idx_v, rows_v, sem = idx_bufs[buf], rows_bufs[buf], sems[buf]
        cid = wid * NCHUNK + g
        pltpu.sync_copy(texts_hbm.at[pl.ds(cid * CHUNK_IDX, CHUNK_IDX)], idx_v)
        for t in range(CHUNK_IDX // 16):
            v = idx_v[pl.ds(16 * t, 16)]
            idx_v[pl.ds(16 * t, 16)] = ((v & PACK_MASK) << 3) | (v >> PACK_SHIFT)
        for c in range(CB):
            for off, ln in IDX_SPLITS:
                pltpu.async_copy(
                    table_hbm.at[idx_v.at[pl.ds(c * HIST + off, ln)]],
                    rows_v.at[pl.ds(c * HIST + off, ln)],
                    sem,
                )

    def drain(buf):
        rows_v, sem = rows_bufs[buf], sems[buf]
        for c in range(CB):
            for off, ln in IDX_SPLITS:
                pltpu.make_async_copy(
                    table_hbm.at[idx_bufs[buf].at[pl.ds(c * HIST + off, ln)]],
                    rows_v.at[pl.ds(c * HIST + off, ln)],
                    sem,
                ).wait()

    def compute(g, buf):
        rows_v = rows_bufs[buf]
        cid = wid * NCHUNK + g
        hi_mask = jnp.full((16,), -65536, jnp.int32)
        for c in range(CB):
            def sum_body(i, carry):
                ae, ao = carry
                r = c * HIST + i * 4
                for u in range(4):
                    w = rows_v[r + u, 0:ROW_I32]
                    ae = ae + plsc.bitcast(w << 16, jnp.float32)
                    ao = ao + plsc.bitcast(w & hi_mask, jnp.float32)
                return ae, ao

            ae, ao = lax.fori_loop(
                0, HIST // 4, sum_body,
                (jnp.zeros((16,), jnp.float32), jnp.zeros((16,), jnp.float32)),
            )
            acc_v[c, 0:16] = ae * inv
            acc_v[c, 16:32] = ao * inv
        pltpu.sync_copy(acc_v, out_hbm.at[pl.ds(cid * CB, CB)])

    fire(0, 0)

    def pair_body(g2, _):
        g = g2 * 2

        @pl.when(g + 1 < NCHUNK)
        def _():
            fire(g + 1, 1)

        drain(0)
        compute(g, 0)

        @pl.when(g + 2 < NCHUNK)
        def _():
            fire(g + 2, 0)

        drain(1)
        compute(g + 1, 1)
        return ()

    lax.fori_loop(0, NCHUNK // 2, pair_body, ())


def _sc_pool(texts_flat, table_view):
    mesh = plsc.VectorSubcoreMesh(core_axis_name="c", subcore_axis_name="s")
    f = pl.kernel(
        _pool_body,
        mesh=mesh,
        compiler_params=pltpu.CompilerParams(
            use_tc_tiling_on_sc=False, needs_layout_passes=False),
        out_type=jax.ShapeDtypeStruct((BATCH, EMBED), jnp.float32),
        scratch_types=[
            pltpu.VMEM((CHUNK_IDX,), jnp.int32),
            pltpu.VMEM((CHUNK_IDX,), jnp.int32),
            pltpu.VMEM((CHUNK_IDX, ROW_I32), jnp.int32),
            pltpu.VMEM((CHUNK_IDX, ROW_I32), jnp.int32),
            pltpu.VMEM((CB, EMBED), jnp.float32),
            pltpu.SemaphoreType.DMA,
            pltpu.SemaphoreType.DMA,
        ],
    )
    return f(texts_flat, table_view)


RL_VB = 2048
RL_QBLK = PACK_S // RL_VB
RL_INBLKS = (VOCAB + RL_VB - 1) // RL_VB


def _relayout_body(*refs):
    slices = [r[...] for r in refs[:PACK_SLOTS]]
    out_ref = refs[PACK_SLOTS]
    x8 = jnp.concatenate(slices, axis=0)
    y = jnp.transpose(x8, (1, 0))
    yr = y.astype(jnp.bfloat16).astype(jnp.float32)
    yi = jax.lax.bitcast_convert_type(yr, jnp.int32)
    for a in range(PACK_SLOTS):
        lo = lax.shift_right_logical(yi[:, 32 * a:32 * a + 16], 16)
        hi = yi[:, 32 * a + 16:32 * a + 32]
        out_ref[:, ROW_I32 * a:ROW_I32 * (a + 1)] = lo | hi


def _slot_spec(a):
    def imap(i):
        return (0, jnp.minimum(a * RL_QBLK + i, RL_INBLKS - 1))
    return pl.BlockSpec((EMBED, RL_VB), imap)


def _tc_relayout(table):
    packed = pl.pallas_call(
        _relayout_body,
        grid=(RL_QBLK,),
        in_specs=[_slot_spec(a) for a in range(PACK_SLOTS)],
        out_specs=pl.BlockSpec((RL_VB, PACK_SLOTS * ROW_I32), lambda i: (i, 0)),
        out_shape=jax.ShapeDtypeStruct(
            (PACK_S, PACK_SLOTS * ROW_I32), jnp.int32),
    )(*([table.T] * PACK_SLOTS))
    return packed.reshape(VIEW_ROWS, ROW_I32)


def _mlp_body(x_ref, w1_ref, b1_ref, w2_ref, b2_ref, o_ref):
    h = jnp.dot(x_ref[...], w1_ref[...], preferred_element_type=jnp.float32)
    h = jnp.maximum(h + b1_ref[...], 0.0)
    o = jnp.dot(h, w2_ref[...], preferred_element_type=jnp.float32)
    o_ref[...] = o + b2_ref[...]


def _tc_mlp(pooled, W1, b1, W2, b2):
    BM = 512
    grid = (BATCH // BM,)
    return pl.pallas_call(
        _mlp_body,
        grid=grid,
        in_specs=[
            pl.BlockSpec((BM, EMBED), lambda i: (i, 0)),
            pl.BlockSpec((EMBED, HIDDEN), lambda i: (0, 0)),
            pl.BlockSpec((1, HIDDEN), lambda i: (0, 0)),
            pl.BlockSpec((HIDDEN, OUT_VOCAB), lambda i: (0, 0)),
            pl.BlockSpec((1, OUT_VOCAB), lambda i: (0, 0)),
        ],
        out_specs=pl.BlockSpec((BM, OUT_VOCAB), lambda i: (i, 0)),
        out_shape=jax.ShapeDtypeStruct((BATCH, OUT_VOCAB), jnp.float32),
    )(pooled, W1, b1.reshape(1, HIDDEN), W2, b2.reshape(1, OUT_VOCAB))


def kernel(texts, table, W1, b1, W2, b2):
    table_view = _tc_relayout(table)
    texts_flat = texts.astype(jnp.int32).reshape(BATCH * HIST)
    pooled = _sc_pool(texts_flat, table_view)
    return _tc_mlp(pooled, W1, b1, W2, b2)


# speedup vs baseline: 2.2834x; 1.5045x over previous
import functools

import jax
import jax.numpy as jnp
from jax import lax
from jax.experimental import pallas as pl
from jax.experimental.pallas import tpu as pltpu
from jax.experimental.pallas import tpu_sc as plsc

VOCAB = 1000000
EMBED = 32
HIDDEN = 64
OUT_VOCAB = 1000
BATCH = 4096
HIST = 200

PACK_S = 1 << 17
PACK_SHIFT = 17
PACK_MASK = PACK_S - 1
PACK_SLOTS = 8
VIEW_ROWS = PACK_SLOTS * PACK_S
ROW_I32 = EMBED // 2

NC = 2
NS = 16
NW = NC * NS
B_PER_W = BATCH // NW
CB = 4
NCHUNK = B_PER_W // CB
CHUNK_IDX = CB * HIST
IDX_SPLITS = ((0, 128), (128, 72))


def _pool_body(texts_hbm, table_hbm, out_hbm,
               idx0, idx1, rows0, rows1, acc_v, sem0, sem1):
    wid = lax.axis_index("s") * NC + lax.axis_index("c")
    inv = jnp.float32(1.0 / HIST)
    idx_bufs = (idx0, idx1)
    rows_bufs = (rows0, rows1)
    sems = (sem0, sem1)

    def fire(g, buf):
        idx_v, rows_v, sem = idx_bufs[buf], rows_bufs[buf], sems[buf]
        cid = wid * NCHUNK + g
        pltpu.sync_copy(texts_hbm.at[pl.ds(cid * CHUNK_IDX, CHUNK_IDX)], idx_v)
        for t in range(CHUNK_IDX // 16):
            v = idx_v[pl.ds(16 * t, 16)]
            idx_v[pl.ds(16 * t, 16)] = ((v & PACK_MASK) << 3) | (v >> PACK_SHIFT)
        for c in range(CB):
            for off, ln in IDX_SPLITS:
                pltpu.async_copy(
                    table_hbm.at[idx_v.at[pl.ds(c * HIST + off, ln)]],
                    rows_v.at[pl.ds(c * HIST + off, ln)],
                    sem,
                )

    def drain(buf):
        rows_v, sem = rows_bufs[buf], sems[buf]
        for c in range(CB):
            for off, ln in IDX_SPLITS:
                pltpu.make_async_copy(
                    table_hbm.at[idx_bufs[buf].at[pl.ds(c * HIST + off, ln)]],
                    rows_v.at[pl.ds(c * HIST + off, ln)],
                    sem,
                ).wait()

    def compute(g, buf):
        rows_v = rows_bufs[buf]
        cid = wid * NCHUNK + g
        hi_mask = jnp.full((16,), -65536, jnp.int32)
        for c in range(CB):
            def sum_body(i, carry):
                ae, ao = carry
                r = c * HIST + i * 4
                for u in range(4):
                    w = rows_v[r + u, 0:ROW_I32]
                    ae = ae + plsc.bitcast(w << 16, jnp.float32)
                    ao = ao + plsc.bitcast(w & hi_mask, jnp.float32)
                return ae, ao

            ae, ao = lax.fori_loop(
                0, HIST // 4, sum_body,
                (jnp.zeros((16,), jnp.float32), jnp.zeros((16,), jnp.float32)),
            )
            acc_v[c, 0:16] = ae * inv
            acc_v[c, 16:32] = ao * inv
        pltpu.sync_copy(acc_v, out_hbm.at[pl.ds(cid * CB, CB)])

    fire(0, 0)

    def pair_body(g2, _):
        g = g2 * 2

        @pl.when(g + 1 < NCHUNK)
        def _():
            fire(g + 1, 1)

        drain(0)
        compute(g, 0)

        @pl.when(g + 2 < NCHUNK)
        def _():
            fire(g + 2, 0)

        drain(1)
        compute(g + 1, 1)
        return ()

    lax.fori_loop(0, NCHUNK // 2, pair_body, ())


def _sc_pool(texts_flat, table_view):
    mesh = plsc.VectorSubcoreMesh(core_axis_name="c", subcore_axis_name="s")
    f = pl.kernel(
        _pool_body,
        mesh=mesh,
        compiler_params=pltpu.CompilerParams(
            use_tc_tiling_on_sc=False, needs_layout_passes=False),
        out_type=jax.ShapeDtypeStruct((BATCH, EMBED), jnp.float32),
        scratch_types=[
            pltpu.VMEM((CHUNK_IDX,), jnp.int32),
            pltpu.VMEM((CHUNK_IDX,), jnp.int32),
            pltpu.VMEM((CHUNK_IDX, ROW_I32), jnp.int32),
            pltpu.VMEM((CHUNK_IDX, ROW_I32), jnp.int32),
            pltpu.VMEM((CB, EMBED), jnp.float32),
            pltpu.SemaphoreType.DMA,
            pltpu.SemaphoreType.DMA,
        ],
    )
    return f(texts_flat, table_view)


RL_VB = 2048
RL_QBLK = PACK_S // RL_VB
RL_INBLKS = (VOCAB + RL_VB - 1) // RL_VB


def _relayout_body(*refs):
    out_ref = refs[PACK_SLOTS]
    words = []
    for a in range(PACK_SLOTS):
        x = refs[a][...]
        xr = x.astype(jnp.bfloat16).astype(jnp.float32)
        xi = jax.lax.bitcast_convert_type(xr, jnp.int32)
        lo = lax.shift_right_logical(xi[0:ROW_I32, :], 16)
        hi = xi[ROW_I32:EMBED, :]
        words.append(lo | hi)
    w = jnp.concatenate(words, axis=0)
    out_ref[...] = jnp.transpose(w, (1, 0))


def _slot_spec(a):
    def imap(i):
        return (0, jnp.minimum(a * RL_QBLK + i, RL_INBLKS - 1))
    return pl.BlockSpec((EMBED, RL_VB), imap)


def _tc_relayout(table):
    packed = pl.pallas_call(
        _relayout_body,
        grid=(RL_QBLK,),
        in_specs=[_slot_spec(a) for a in range(PACK_SLOTS)],
        out_specs=pl.BlockSpec((RL_VB, PACK_SLOTS * ROW_I32), lambda i: (i, 0)),
        out_shape=jax.ShapeDtypeStruct(
            (PACK_S, PACK_SLOTS * ROW_I32), jnp.int32),
    )(*([table.T] * PACK_SLOTS))
    return packed.reshape(VIEW_ROWS, ROW_I32)


def _mlp_body(x_ref, w1_ref, b1_ref, w2_ref, b2_ref, o_ref):
    h = jnp.dot(x_ref[...], w1_ref[...], preferred_element_type=jnp.float32)
    h = jnp.maximum(h + b1_ref[...], 0.0)
    o = jnp.dot(h, w2_ref[...], preferred_element_type=jnp.float32)
    o_ref[...] = o + b2_ref[...]


def _tc_mlp(pooled, W1, b1, W2, b2):
    BM = 512
    grid = (BATCH // BM,)
    return pl.pallas_call(
        _mlp_body,
        grid=grid,
        in_specs=[
            pl.BlockSpec((BM, EMBED), lambda i: (i, 0)),
            pl.BlockSpec((EMBED, HIDDEN), lambda i: (0, 0)),
            pl.BlockSpec((1, HIDDEN), lambda i: (0, 0)),
            pl.BlockSpec((HIDDEN, OUT_VOCAB), lambda i: (0, 0)),
            pl.BlockSpec((1, OUT_VOCAB), lambda i: (0, 0)),
        ],
        out_specs=pl.BlockSpec((BM, OUT_VOCAB), lambda i: (i, 0)),
        out_shape=jax.ShapeDtypeStruct((BATCH, OUT_VOCAB), jnp.float32),
    )(pooled, W1, b1.reshape(1, HIDDEN), W2, b2.reshape(1, OUT_VOCAB))


def kernel(texts, table, W1, b1, W2, b2):
    table_view = _tc_relayout(table)
    texts_flat = texts.astype(jnp.int32).reshape(BATCH * HIST)
    pooled = _sc_pool(texts_flat, table_view)
    return _tc_mlp(pooled, W1, b1, W2, b2)


# speedup vs baseline: 2.5774x; 1.1288x over previous
import functools

import jax
import jax.numpy as jnp
from jax import lax
from jax.experimental import pallas as pl
from jax.experimental.pallas import tpu as pltpu
from jax.experimental.pallas import tpu_sc as plsc

VOCAB = 1000000
EMBED = 32
HIDDEN = 64
OUT_VOCAB = 1000
BATCH = 4096
HIST = 200

PACK_S = 1 << 17
PACK_SHIFT = 17
PACK_MASK = PACK_S - 1
PACK_SLOTS = 8
VIEW_ROWS = PACK_SLOTS * PACK_S
ROW_I32 = EMBED // 2

NC = 2
NS = 16
NW = NC * NS
B_PER_W = BATCH // NW
CB = 8
NCHUNK = B_PER_W // CB
CHUNK_IDX = CB * HIST
IDX_SPLITS = ((0, 128), (128, 72))


def _pool_body(texts_hbm, table_hbm, out_hbm,
               idx0, idx1, rows0, rows1, acc_v, sem0, sem1):
    wid = lax.axis_index("s") * NC + lax.axis_index("c")
    inv = jnp.float32(1.0 / HIST)
    idx_bufs = (idx0, idx1)
    rows_bufs = (rows0, rows1)
    sems = (sem0, sem1)

    def fire(g, buf):
        idx_v, rows_v, sem = idx_bufs[buf], rows_bufs[buf], sems[buf]
        cid = wid * NCHUNK + g
        pltpu.sync_copy(texts_hbm.at[pl.ds(cid * CHUNK_IDX, CHUNK_IDX)], idx_v)
        for t in range(CHUNK_IDX // 16):
            v = idx_v[pl.ds(16 * t, 16)]
            idx_v[pl.ds(16 * t, 16)] = ((v & PACK_MASK) << 3) | (v >> PACK_SHIFT)
        for c in range(CB):
            for off, ln in IDX_SPLITS:
                pltpu.async_copy(
                    table_hbm.at[idx_v.at[pl.ds(c * HIST + off, ln)]],
                    rows_v.at[pl.ds(c * HIST + off, ln)],
                    sem,
                )

    def drain(buf):
        rows_v, sem = rows_bufs[buf], sems[buf]
        for c in range(CB):
            for off, ln in IDX_SPLITS:
                pltpu.make_async_copy(
                    table_hbm.at[idx_bufs[buf].at[pl.ds(c * HIST + off, ln)]],
                    rows_v.at[pl.ds(c * HIST + off, ln)],
                    sem,
                ).wait()

    def compute(g, buf):
        rows_v = rows_bufs[buf]
        cid = wid * NCHUNK + g
        hi_mask = jnp.full((16,), -65536, jnp.int32)
        for c in range(CB):
            def sum_body(i, carry):
                ae, ao = carry
                r = c * HIST + i * 4
                for u in range(4):
                    w = rows_v[r + u, 0:ROW_I32]
                    ae = ae + plsc.bitcast(w << 16, jnp.float32)
                    ao = ao + plsc.bitcast(w & hi_mask, jnp.float32)
                return ae, ao

            ae, ao = lax.fori_loop(
                0, HIST // 4, sum_body,
                (jnp.zeros((16,), jnp.float32), jnp.zeros((16,), jnp.float32)),
            )
            acc_v[c, 0:16] = ae * inv
            acc_v[c, 16:32] = ao * inv
        pltpu.sync_copy(acc_v, out_hbm.at[pl.ds(cid * CB, CB)])

    fire(0, 0)

    def pair_body(g2, _):
        g = g2 * 2

        @pl.when(g + 1 < NCHUNK)
        def _():
            fire(g + 1, 1)

        drain(0)
        compute(g, 0)

        @pl.when(g + 2 < NCHUNK)
        def _():
            fire(g + 2, 0)

        drain(1)
        compute(g + 1, 1)
        return ()

    lax.fori_loop(0, NCHUNK // 2, pair_body, ())


def _sc_pool(texts_flat, table_view):
    mesh = plsc.VectorSubcoreMesh(core_axis_name="c", subcore_axis_name="s")
    f = pl.kernel(
        _pool_body,
        mesh=mesh,
        compiler_params=pltpu.CompilerParams(
            use_tc_tiling_on_sc=False, needs_layout_passes=False),
        out_type=jax.ShapeDtypeStruct((BATCH, EMBED), jnp.float32),
        scratch_types=[
            pltpu.VMEM((CHUNK_IDX,), jnp.int32),
            pltpu.VMEM((CHUNK_IDX,), jnp.int32),
            pltpu.VMEM((CHUNK_IDX, ROW_I32), jnp.int32),
            pltpu.VMEM((CHUNK_IDX, ROW_I32), jnp.int32),
            pltpu.VMEM((CB, EMBED), jnp.float32),
            pltpu.SemaphoreType.DMA,
            pltpu.SemaphoreType.DMA,
        ],
    )
    return f(texts_flat, table_view)


RL_VB = 2048
RL_QBLK = PACK_S // RL_VB
RL_INBLKS = (VOCAB + RL_VB - 1) // RL_VB


def _relayout_body(*refs):
    out_ref = refs[PACK_SLOTS]
    words = []
    for a in range(PACK_SLOTS):
        x = refs[a][...]
        xr = x.astype(jnp.bfloat16).astype(jnp.float32)
        xi = jax.lax.bitcast_convert_type(xr, jnp.int32)
        lo = lax.shift_right_logical(xi[0:ROW_I32, :], 16)
        hi = xi[ROW_I32:EMBED, :]
        words.append(lo | hi)
    w = jnp.concatenate(words, axis=0)
    out_ref[...] = jnp.transpose(w, (1, 0))


def _slot_spec(a):
    def imap(i):
        return (0, jnp.minimum(a * RL_QBLK + i, RL_INBLKS - 1))
    return pl.BlockSpec((EMBED, RL_VB), imap)


def _tc_relayout(table):
    packed = pl.pallas_call(
        _relayout_body,
        grid=(RL_QBLK,),
        in_specs=[_slot_spec(a) for a in range(PACK_SLOTS)],
        out_specs=pl.BlockSpec((RL_VB, PACK_SLOTS * ROW_I32), lambda i: (i, 0)),
        out_shape=jax.ShapeDtypeStruct(
            (PACK_S, PACK_SLOTS * ROW_I32), jnp.int32),
    )(*([table.T] * PACK_SLOTS))
    return packed.reshape(VIEW_ROWS, ROW_I32)


def _mlp_body(x_ref, w1_ref, b1_ref, w2t_ref, b2_ref, o_ref):
    h = jnp.dot(x_ref[...], w1_ref[...], preferred_element_type=jnp.float32)
    h = jnp.maximum(h + b1_ref[...], 0.0)
    ht = jnp.transpose(h, (1, 0))
    o = jnp.dot(w2t_ref[...], ht, preferred_element_type=jnp.float32)
    o_ref[...] = o + b2_ref[...]


def _tc_mlp(pooled, W1, b1, W2, b2):
    BM = 512
    grid = (BATCH // BM,)
    logits_t = pl.pallas_call(
        _mlp_body,
        grid=grid,
        in_specs=[
            pl.BlockSpec((BM, EMBED), lambda i: (i, 0)),
            pl.BlockSpec((EMBED, HIDDEN), lambda i: (0, 0)),
            pl.BlockSpec((1, HIDDEN), lambda i: (0, 0)),
            pl.BlockSpec((OUT_VOCAB, HIDDEN), lambda i: (0, 0)),
            pl.BlockSpec((OUT_VOCAB, 1), lambda i: (0, 0)),
        ],
        out_specs=pl.BlockSpec((OUT_VOCAB, BM), lambda i: (0, i)),
        out_shape=jax.ShapeDtypeStruct((OUT_VOCAB, BATCH), jnp.float32),
    )(pooled, W1, b1.reshape(1, HIDDEN), W2.T, b2.reshape(OUT_VOCAB, 1))
    return logits_t.T


def kernel(texts, table, W1, b1, W2, b2):
    table_view = _tc_relayout(table)
    texts_flat = texts.astype(jnp.int32).reshape(BATCH * HIST)
    pooled = _sc_pool(texts_flat, table_view)
    return _tc_mlp(pooled, W1, b1, W2, b2)


# speedup vs baseline: 2.5864x; 1.0035x over previous
import jax
import jax.numpy as jnp
from jax import lax
from jax.experimental import pallas as pl
from jax.experimental.pallas import tpu as pltpu
from jax.experimental.pallas import tpu_sc as plsc

VOCAB = 1000000
EMBED = 32
HIDDEN = 64
OUT_VOCAB = 1000
BATCH = 4096
HIST = 200

PACK_S = 1 << 17
PACK_SHIFT = 17
PACK_MASK = PACK_S - 1
PACK_SLOTS = 8
VIEW_ROWS = PACK_SLOTS * PACK_S
ROW_I32 = EMBED // 2

NC = 2
NS = 16
NW = NC * NS
B_PER_W = BATCH // NW
CB = 8
NCHUNK = B_PER_W // CB
CHUNK_IDX = CB * HIST
IDX_SPLITS = ((0, 128), (128, 72))


def _pool_body(texts_hbm, table_hbm, out_hbm,
               idx0, idx1, rows0, rows1, acc_v, sem0, sem1):
    wid = lax.axis_index("s") * NC + lax.axis_index("c")
    inv = jnp.float32(1.0 / HIST)
    idx_bufs = (idx0, idx1)
    rows_bufs = (rows0, rows1)
    sems = (sem0, sem1)

    def fire(g, buf):
        idx_v, rows_v, sem = idx_bufs[buf], rows_bufs[buf], sems[buf]
        cid = wid * NCHUNK + g
        pltpu.sync_copy(texts_hbm.at[pl.ds(cid * CHUNK_IDX, CHUNK_IDX)], idx_v)
        for t in range(CHUNK_IDX // 16):
            v = idx_v[pl.ds(16 * t, 16)]
            idx_v[pl.ds(16 * t, 16)] = ((v & PACK_MASK) << 3) | (v >> PACK_SHIFT)
        for c in range(CB):
            for off, ln in IDX_SPLITS:
                pltpu.async_copy(
                    table_hbm.at[idx_v.at[pl.ds(c * HIST + off, ln)]],
                    rows_v.at[pl.ds(c * HIST + off, ln)],
                    sem,
                )

    def drain(buf):
        rows_v, sem = rows_bufs[buf], sems[buf]
        for c in range(CB):
            for off, ln in IDX_SPLITS:
                pltpu.make_async_copy(
                    table_hbm.at[idx_bufs[buf].at[pl.ds(c * HIST + off, ln)]],
                    rows_v.at[pl.ds(c * HIST + off, ln)],
                    sem,
                ).wait()

    def compute(g, buf):
        rows_v = rows_bufs[buf]
        cid = wid * NCHUNK + g
        hi_mask = jnp.full((16,), -65536, jnp.int32)
        for c in range(CB):
            def sum_body(i, carry):
                ae, ao = carry
                r = c * HIST + i * 4
                for u in range(4):
                    w = rows_v[r + u, 0:ROW_I32]
                    ae = ae + plsc.bitcast(w << 16, jnp.float32)
                    ao = ao + plsc.bitcast(w & hi_mask, jnp.float32)
                return ae, ao

            ae, ao = lax.fori_loop(
                0, HIST // 4, sum_body,
                (jnp.zeros((16,), jnp.float32), jnp.zeros((16,), jnp.float32)),
            )
            acc_v[c, 0:16] = ae * inv
            acc_v[c, 16:32] = ao * inv
        pltpu.sync_copy(acc_v, out_hbm.at[pl.ds(cid * CB, CB)])

    fire(0, 0)

    def pair_body(g2, _):
        g = g2 * 2

        @pl.when(g + 1 < NCHUNK)
        def _():
            fire(g + 1, 1)

        drain(0)
        compute(g, 0)

        @pl.when(g + 2 < NCHUNK)
        def _():
            fire(g + 2, 0)

        drain(1)
        compute(g + 1, 1)
        return ()

    lax.fori_loop(0, NCHUNK // 2, pair_body, ())


def _sc_pool(texts_flat, table_view):
    mesh = plsc.VectorSubcoreMesh(core_axis_name="c", subcore_axis_name="s")
    f = pl.kernel(
        _pool_body,
        mesh=mesh,
        compiler_params=pltpu.CompilerParams(
            use_tc_tiling_on_sc=False, needs_layout_passes=False),
        out_type=jax.ShapeDtypeStruct((BATCH, EMBED), jnp.float32),
        scratch_types=[
            pltpu.VMEM((CHUNK_IDX,), jnp.int32),
            pltpu.VMEM((CHUNK_IDX,), jnp.int32),
            pltpu.VMEM((CHUNK_IDX, ROW_I32), jnp.int32),
            pltpu.VMEM((CHUNK_IDX, ROW_I32), jnp.int32),
            pltpu.VMEM((CB, EMBED), jnp.float32),
            pltpu.SemaphoreType.DMA,
            pltpu.SemaphoreType.DMA,
        ],
    )
    return f(texts_flat, table_view)


RL_VB = 2048
RL_QBLK = PACK_S // RL_VB
RL_INBLKS = (VOCAB + RL_VB - 1) // RL_VB


def _relayout_body(*refs):
    out_ref = refs[PACK_SLOTS]
    words = []
    for a in range(PACK_SLOTS):
        x = refs[a][...]
        xr = x.astype(jnp.bfloat16).astype(jnp.float32)
        xi = jax.lax.bitcast_convert_type(xr, jnp.int32)
        lo = lax.shift_right_logical(xi[0:ROW_I32, :], 16)
        hi = xi[ROW_I32:EMBED, :]
        words.append(lo | hi)
    w = jnp.concatenate(words, axis=0)
    out_ref[...] = jnp.transpose(w, (1, 0))


def _slot_spec(a):
    def imap(i):
        return (0, jnp.minimum(a * RL_QBLK + i, RL_INBLKS - 1))
    return pl.BlockSpec((EMBED, RL_VB), imap)


def _tc_relayout(table):
    packed = pl.pallas_call(
        _relayout_body,
        grid=(RL_QBLK,),
        in_specs=[_slot_spec(a) for a in range(PACK_SLOTS)],
        out_specs=pl.BlockSpec((RL_VB, PACK_SLOTS * ROW_I32), lambda i: (i, 0)),
        out_shape=jax.ShapeDtypeStruct(
            (PACK_S, PACK_SLOTS * ROW_I32), jnp.int32),
    )(*([table.T] * PACK_SLOTS))
    return packed.reshape(VIEW_ROWS, ROW_I32)


def _mlp_body(x_ref, w1_ref, b1_ref, w2t_ref, b2_ref, o_ref):
    h = jnp.dot(x_ref[...], w1_ref[...], preferred_element_type=jnp.float32)
    h = jnp.maximum(h + b1_ref[...], 0.0)
    ht = jnp.transpose(h, (1, 0))
    o = jnp.dot(w2t_ref[...], ht, preferred_element_type=jnp.float32)
    o_ref[...] = o + b2_ref[...]


def _tc_mlp(pooled, W1, b1, W2, b2):
    BM = 512
    grid = (BATCH // BM,)
    logits_t = pl.pallas_call(
        _mlp_body,
        grid=grid,
        in_specs=[
            pl.BlockSpec((BM, EMBED), lambda i: (i, 0)),
            pl.BlockSpec((EMBED, HIDDEN), lambda i: (0, 0)),
            pl.BlockSpec((1, HIDDEN), lambda i: (0, 0)),
            pl.BlockSpec((OUT_VOCAB, HIDDEN), lambda i: (0, 0)),
            pl.BlockSpec((OUT_VOCAB, 1), lambda i: (0, 0)),
        ],
        out_specs=pl.BlockSpec((OUT_VOCAB, BM), lambda i: (0, i)),
        out_shape=jax.ShapeDtypeStruct((OUT_VOCAB, BATCH), jnp.float32),
    )(pooled, W1, b1.reshape(1, HIDDEN), W2.T, b2.reshape(OUT_VOCAB, 1))
    return logits_t.T


def kernel(texts, table, W1, b1, W2, b2):
    table_view = _tc_relayout(table)
    texts_flat = texts.astype(jnp.int32).reshape(BATCH * HIST)
    pooled = _sc_pool(texts_flat, table_view)
    return _tc_mlp(pooled, W1, b1, W2, b2)
